# baseline (speedup 1.0000x reference)
import functools

import jax
import jax.numpy as jnp
from jax import lax
from jax.experimental import pallas as pl
from jax.experimental.pallas import tpu as pltpu
from jax.experimental.pallas import tpu_sc as plsc

N = 5000
D = 128
N_PAD = 5120
E = 160000
L = 50000
NW = 32
ET = E // 16
EC = 80
LC = 13
L_PAD = NW * LC * 128
LW = LC * 128

@functools.cache
def _get_mesh():
    return plsc.VectorSubcoreMesh(core_axis_name="c", subcore_axis_name="s")


@functools.cache
def _build_sc_edge_agg():
  kern = functools.partial(
    pl.kernel,
    out_type=(
        jax.ShapeDtypeStruct((N_PAD, D), jnp.float32),
        jax.ShapeDtypeStruct((16, N_PAD), jnp.int32),
        jax.ShapeDtypeStruct((N_PAD, D), jnp.float32),
        jax.ShapeDtypeStruct((16, N_PAD), jnp.int32),
    ),
    mesh=_get_mesh(),
    compiler_params=pltpu.CompilerParams(needs_layout_passes=False),
    scratch_types=[
        pltpu.VMEM_SHARED((N_PAD, D), jnp.float32),
        pltpu.VMEM((EC, 128), jnp.int32),
        pltpu.VMEM((EC, 128), jnp.int32),
        pltpu.VMEM((2, 128, D), jnp.float32),
        pltpu.VMEM((N_PAD,), jnp.int32),
        pltpu.VMEM((16, D), jnp.float32),
        pltpu.SemaphoreType.DMA,
        pltpu.SemaphoreType.DMA,
    ],
  )

  @kern
  def _sc_edge_agg(xv_hbm, xh_hbm, svh, dvh, shv, dhv,
                   out_svh, out_cvh, out_shv, out_chv,
                   acc, idx_s, idx_d, rows, hist, zbuf, sem0, sem1):
      cid = lax.axis_index("c")
      sid = lax.axis_index("s")
      sems = (sem0, sem1)

      zero16f = jnp.zeros((16,), jnp.float32)
      zero16i = jnp.zeros((16,), jnp.int32)
      one16i = jnp.full((16,), 1, jnp.int32)

      for r in range(16):
          for c8 in range(D // 16):
              zbuf[r, pl.ds(c8 * 16, 16)] = zero16f

      @pl.loop(0, N_PAD // 16)
      def _zero_hist(i):
          hist[pl.ds(pl.multiple_of(i * 16, 16), 16)] = zero16i

      row0 = sid * 320
      for k in range(20):
          pltpu.sync_copy(zbuf, acc.at[pl.ds(row0 + k * 16, 16)])
      plsc.subcore_barrier()

      def run_edges(x_hbm, s_hbm, d_hbm, cnt_out):
          pltpu.sync_copy(s_hbm.at[sid], idx_s)
          pltpu.sync_copy(d_hbm.at[sid], idx_d)
          for b in range(2):
              pltpu.async_copy(x_hbm.at[idx_s.at[b]], rows.at[b], sems[b])

          @pl.loop(0, EC, step=2)
          def _chunks(j0):
              for b in range(2):
                  j = j0 + b
                  pltpu.make_async_copy(
                      x_hbm.at[idx_s.at[j]], rows.at[b], sems[b]).wait()
                  pltpu.sync_copy(rows.at[b], acc.at[idx_d.at[j]], add=True)

                  @pl.when(j0 < EC - 2)
                  def _():
                      pltpu.async_copy(
                          x_hbm.at[idx_s.at[j + 2]], rows.at[b], sems[b])

                  for g in range(8):
                      d16 = idx_d[j, pl.ds(g * 16, 16)]
                      plsc.addupdate_scatter(hist, [d16], one16i)

          pltpu.sync_copy(hist, cnt_out.at[sid])

      @pl.when(cid == 0)
      def _():
          run_edges(xv_hbm, svh, dvh, out_cvh)

      @pl.when(cid == 1)
      def _():
          run_edges(xh_hbm, shv, dhv, out_chv)

      plsc.subcore_barrier()

      @pl.when(cid == 0)
      def _():
          pltpu.sync_copy(acc.at[pl.ds(row0, 320)], out_svh.at[pl.ds(row0, 320)])

      @pl.when(cid == 1)
      def _():
          pltpu.sync_copy(acc.at[pl.ds(row0, 320)], out_shv.at[pl.ds(row0, 320)])

  return _sc_edge_agg


def _tc_combine_body(svh_ref, cvh_ref, shv_ref, chv_ref, xv_ref, xh_ref,
                     wlt_vh_ref, wrt_vh_ref, bvh_ref,
                     wlt_hv_ref, wrt_hv_ref, bhv_ref,
                     w1t_ref, zv1_ref, zh1_ref):
    ones_j = jnp.ones((16, D), jnp.float32)

    def node_update(sums, cnts, x_dst, wlt, wrt, b):
        cnt2d = lax.dot_general(cnts.astype(jnp.float32), ones_j,
                                (((0,), (0,)), ((), ())),
                                preferred_element_type=jnp.float32)
        mean = sums[:N] / jnp.maximum(cnt2d[:N], 1.0)
        h = (jnp.dot(mean, wlt, preferred_element_type=jnp.float32)
             + b
             + jnp.dot(x_dst, wrt, preferred_element_type=jnp.float32))
        return jnp.maximum(h, 0.0)

    z_host = node_update(svh_ref[...], cvh_ref[...], xh_ref[...],
                         wlt_vh_ref[...], wrt_vh_ref[...], bvh_ref[...])
    z_virus = node_update(shv_ref[...], chv_ref[...], xv_ref[...],
                          wlt_hv_ref[...], wrt_hv_ref[...], bhv_ref[...])
    w1t = w1t_ref[...]
    zv1_ref[...] = jnp.dot(z_virus, w1t, preferred_element_type=jnp.float32)
    zh1_ref[...] = jnp.dot(z_host, w1t, preferred_element_type=jnp.float32)


_tc_combine = pl.pallas_call(
    _tc_combine_body,
    compiler_params=pltpu.CompilerParams(fuse_transposed_lhs_in_matmul=True),
    out_shape=(
        jax.ShapeDtypeStruct((N, D), jnp.float32),
        jax.ShapeDtypeStruct((N, D), jnp.float32),
    ),
)


@functools.cache
def _build_sc_decoder_gather():
  kern = functools.partial(
    pl.kernel,
    out_type=(
        jax.ShapeDtypeStruct((L_PAD, D), jnp.float32),
        jax.ShapeDtypeStruct((L_PAD, D), jnp.float32),
    ),
    mesh=_get_mesh(),
    scratch_types=[
        pltpu.VMEM((128,), jnp.int32),
        pltpu.VMEM((128,), jnp.int32),
        pltpu.VMEM((128, D), jnp.float32),
        pltpu.VMEM((128, D), jnp.float32),
        pltpu.SemaphoreType.DMA,
        pltpu.SemaphoreType.DMA,
    ],
  )

  @kern
  def _sc_decoder_gather(zv1_hbm, zh1_hbm, row_hbm, col_hbm,
                         out_a, out_b,
                         idx_r, idx_c, ra, rb, sem_a, sem_b):
      cid = lax.axis_index("c")
      sid = lax.axis_index("s")
      wid = sid * 2 + cid
      base0 = wid * LW

      @pl.loop(0, LC)
      def _chunk(j):
          base = pl.multiple_of(base0 + j * 128, 8)
          pltpu.sync_copy(row_hbm.at[pl.ds(base, 128)], idx_r)
          pltpu.sync_copy(col_hbm.at[pl.ds(base, 128)], idx_c)
          cp_a = pltpu.async_copy(zv1_hbm.at[idx_r], ra, sem_a)
          cp_b = pltpu.async_copy(zh1_hbm.at[idx_c], rb, sem_b)
          cp_a.wait()
          cp_b.wait()
          pltpu.sync_copy(ra, out_a.at[pl.ds(base, 128)])
          pltpu.sync_copy(rb, out_b.at[pl.ds(base, 128)])

  return _sc_decoder_gather


def _tc_decode_body(a_ref, b_ref, b1_ref, w2t_ref, b2_ref, o_ref):
    z = jnp.maximum(a_ref[...] - b_ref[...] + b1_ref[...], 0.0)
    o_ref[...] = (jnp.dot(z, w2t_ref[...], preferred_element_type=jnp.float32)
                  + b2_ref[...])


_tc_decode = pl.pallas_call(
    _tc_decode_body,
    grid=(L_PAD // LW,),
    in_specs=[
        pl.BlockSpec((LW, D), lambda i: (i, 0)),
        pl.BlockSpec((LW, D), lambda i: (i, 0)),
        pl.BlockSpec((1, D), lambda i: (0, 0)),
        pl.BlockSpec((D, 1), lambda i: (0, 0)),
        pl.BlockSpec((1, 1), lambda i: (0, 0)),
    ],
    out_specs=pl.BlockSpec((LW, 1), lambda i: (i, 0)),
    out_shape=jax.ShapeDtypeStruct((L_PAD, 1), jnp.float32),
)


def _tile_edge_idx(a, pad_val):
    a = a.reshape(16, ET)
    pad = jnp.full((16, EC * 128 - ET), pad_val, jnp.int32)
    return jnp.concatenate([a, pad], axis=1).reshape(16, EC, 128)


def _link_idx(a):
    pad = jnp.zeros((L_PAD - L,), jnp.int32)
    return jnp.concatenate([a, pad])


def kernel(x_virus, x_host, edge_index_vh, edge_index_hv, edge_label_index,
           Wl_vh, Wr_vh, b_vh, Wl_hv, Wr_hv, b_hv, W1, b1, W2, b2):
    svh, cvh, shv, chv = _build_sc_edge_agg()(
        x_virus, x_host,
        _tile_edge_idx(edge_index_vh[0], 0),
        _tile_edge_idx(edge_index_vh[1], N),
        _tile_edge_idx(edge_index_hv[0], 0),
        _tile_edge_idx(edge_index_hv[1], N),
    )

    zv1, zh1 = _tc_combine(
        svh, cvh, shv, chv, x_virus, x_host,
        Wl_vh.T, Wr_vh.T, b_vh.reshape(1, D),
        Wl_hv.T, Wr_hv.T, b_hv.reshape(1, D),
        W1.T,
    )

    a, b = _build_sc_decoder_gather()(
        zv1, zh1,
        _link_idx(edge_label_index[0]),
        _link_idx(edge_label_index[1]),
    )

    out = _tc_decode(a, b, b1.reshape(1, D), W2.reshape(D, 1),
                     b2.reshape(1, 1))
    return out[:L, 0]


# speedup vs baseline: 1.1908x; 1.1908x over previous
import functools

import jax
import jax.numpy as jnp
from jax import lax
from jax.experimental import pallas as pl
from jax.experimental.pallas import tpu as pltpu
from jax.experimental.pallas import tpu_sc as plsc

N = 5000
D = 128
N_PAD = 5120
E = 160000
L = 50000
L_PAD = 50176
NW = 32
ET = E // 16
LW = L_PAD // NW


@functools.cache
def _get_mesh():
    return plsc.VectorSubcoreMesh(core_axis_name="c", subcore_axis_name="s")


@functools.cache
def _build_sc_edge_agg():
  kern = functools.partial(
    pl.kernel,
    out_type=(
        jax.ShapeDtypeStruct((N_PAD, D), jnp.float32),
        jax.ShapeDtypeStruct((N_PAD, D), jnp.float32),
        jax.ShapeDtypeStruct((N_PAD, D), jnp.float32),
        jax.ShapeDtypeStruct((N_PAD, D), jnp.float32),
    ),
    mesh=_get_mesh(),
    scratch_types=[
        pltpu.VMEM_SHARED((N_PAD, D), jnp.float32),
        pltpu.VMEM_SHARED((N_PAD, D), jnp.float32),
        pltpu.VMEM((128,), jnp.int32),
        pltpu.VMEM((128,), jnp.int32),
        pltpu.VMEM((16,), jnp.int32),
        pltpu.VMEM((16,), jnp.int32),
        pltpu.VMEM((128, D), jnp.float32),
        pltpu.VMEM((16, D), jnp.float32),
        pltpu.VMEM((128, D), jnp.float32),
        pltpu.VMEM((16, D), jnp.float32),
        pltpu.SemaphoreType.DMA,
    ],
  )

  @kern
  def _sc_edge_agg(xv_hbm, xh_hbm, svh, dvh, shv, dhv,
                   out_svh, out_cvh, out_shv, out_chv,
                   acc, cnt, idx_s, idx_d, idx_s16, idx_d16,
                   rows, rows16, ones, zbuf, sem):
      cid = lax.axis_index("c")
      sid = lax.axis_index("s")

      one16 = jnp.full((16,), 1.0, jnp.float32)
      zero16 = jnp.zeros((16,), jnp.float32)

      @pl.loop(0, 128)
      def _init_ones(r):
          for c8 in range(D // 16):
              ones[r, pl.ds(c8 * 16, 16)] = one16

      for r in range(16):
          for c8 in range(D // 16):
              zbuf[r, pl.ds(c8 * 16, 16)] = zero16
      row0 = sid * 320
      for k in range(20):
          pltpu.sync_copy(zbuf, acc.at[pl.ds(row0 + k * 16, 16)])
          pltpu.sync_copy(zbuf, cnt.at[pl.ds(row0 + k * 16, 16)])
      plsc.subcore_barrier()

      def run_edges(x_hbm, s_hbm, d_hbm):
          base0 = sid * ET

          @pl.loop(0, 78)
          def _chunk(j):
              base = pl.multiple_of(base0 + j * 128, 8)
              pltpu.sync_copy(s_hbm.at[pl.ds(base, 128)], idx_s)
              pltpu.sync_copy(d_hbm.at[pl.ds(base, 128)], idx_d)
              pltpu.async_copy(x_hbm.at[idx_s], rows, sem).wait()
              pltpu.sync_copy(rows, acc.at[idx_d], add=True)
              pltpu.sync_copy(ones, cnt.at[idx_d], add=True)

          tail = base0 + 78 * 128
          pltpu.sync_copy(s_hbm.at[pl.ds(tail, 16)], idx_s16)
          pltpu.sync_copy(d_hbm.at[pl.ds(tail, 16)], idx_d16)
          pltpu.async_copy(x_hbm.at[idx_s16], rows16, sem).wait()
          pltpu.sync_copy(rows16, acc.at[idx_d16], add=True)
          pltpu.sync_copy(ones.at[pl.ds(0, 16)], cnt.at[idx_d16], add=True)

      @pl.when(cid == 0)
      def _():
          run_edges(xv_hbm, svh, dvh)

      @pl.when(cid == 1)
      def _():
          run_edges(xh_hbm, shv, dhv)

      plsc.subcore_barrier()

      @pl.when(cid == 0)
      def _():
          pltpu.sync_copy(acc.at[pl.ds(row0, 320)], out_svh.at[pl.ds(row0, 320)])
          pltpu.sync_copy(cnt.at[pl.ds(row0, 320)], out_cvh.at[pl.ds(row0, 320)])

      @pl.when(cid == 1)
      def _():
          pltpu.sync_copy(acc.at[pl.ds(row0, 320)], out_shv.at[pl.ds(row0, 320)])
          pltpu.sync_copy(cnt.at[pl.ds(row0, 320)], out_chv.at[pl.ds(row0, 320)])

  return _sc_edge_agg


def _tc_combine_body(svh_ref, cvh_ref, shv_ref, chv_ref, xv_ref, xh_ref,
                     wlt_vh_ref, wrt_vh_ref, bvh_ref,
                     wlt_hv_ref, wrt_hv_ref, bhv_ref,
                     w1t_ref, zv1_ref, zh1_ref):
    def node_update(sums, cnts, x_dst, wlt, wrt, b):
        cnt = jnp.maximum(cnts[:N, 0:1], 1.0)
        mean = sums[:N] / cnt
        h = (jnp.dot(mean, wlt, preferred_element_type=jnp.float32)
             + b
             + jnp.dot(x_dst, wrt, preferred_element_type=jnp.float32))
        return jnp.maximum(h, 0.0)

    z_host = node_update(svh_ref[...], cvh_ref[...], xh_ref[...],
                         wlt_vh_ref[...], wrt_vh_ref[...], bvh_ref[...])
    z_virus = node_update(shv_ref[...], chv_ref[...], xv_ref[...],
                          wlt_hv_ref[...], wrt_hv_ref[...], bhv_ref[...])
    w1t = w1t_ref[...]
    zv1_ref[...] = jnp.dot(z_virus, w1t, preferred_element_type=jnp.float32)
    zh1_ref[...] = jnp.dot(z_host, w1t, preferred_element_type=jnp.float32)


_tc_combine = pl.pallas_call(
    _tc_combine_body,
    out_shape=(
        jax.ShapeDtypeStruct((N, D), jnp.float32),
        jax.ShapeDtypeStruct((N, D), jnp.float32),
    ),
)


@functools.cache
def _build_sc_decoder_gather():
  kern = functools.partial(
    pl.kernel,
    out_type=(
        jax.ShapeDtypeStruct((L_PAD, D), jnp.float32),
        jax.ShapeDtypeStruct((L_PAD, D), jnp.float32),
    ),
    mesh=_get_mesh(),
    scratch_types=[
        pltpu.VMEM((128,), jnp.int32),
        pltpu.VMEM((128,), jnp.int32),
        pltpu.VMEM((32,), jnp.int32),
        pltpu.VMEM((32,), jnp.int32),
        pltpu.VMEM((128, D), jnp.float32),
        pltpu.VMEM((128, D), jnp.float32),
        pltpu.VMEM((32, D), jnp.float32),
        pltpu.VMEM((32, D), jnp.float32),
        pltpu.SemaphoreType.DMA,
        pltpu.SemaphoreType.DMA,
    ],
  )

  @kern
  def _sc_decoder_gather(zv1_hbm, zh1_hbm, row_hbm, col_hbm,
                         out_a, out_b,
                         idx_r, idx_c, idx_r32, idx_c32,
                         ra, rb, ra32, rb32, sem_a, sem_b):
      cid = lax.axis_index("c")
      sid = lax.axis_index("s")
      wid = sid * 2 + cid
      base0 = wid * LW

      @pl.loop(0, 12)
      def _chunk(j):
          base = pl.multiple_of(base0 + j * 128, 8)
          pltpu.sync_copy(row_hbm.at[pl.ds(base, 128)], idx_r)
          pltpu.sync_copy(col_hbm.at[pl.ds(base, 128)], idx_c)
          cp_a = pltpu.async_copy(zv1_hbm.at[idx_r], ra, sem_a)
          cp_b = pltpu.async_copy(zh1_hbm.at[idx_c], rb, sem_b)
          cp_a.wait()
          cp_b.wait()
          pltpu.sync_copy(ra, out_a.at[pl.ds(base, 128)])
          pltpu.sync_copy(rb, out_b.at[pl.ds(base, 128)])

      tail = base0 + 12 * 128
      pltpu.sync_copy(row_hbm.at[pl.ds(tail, 32)], idx_r32)
      pltpu.sync_copy(col_hbm.at[pl.ds(tail, 32)], idx_c32)
      cp_a = pltpu.async_copy(zv1_hbm.at[idx_r32], ra32, sem_a)
      cp_b = pltpu.async_copy(zh1_hbm.at[idx_c32], rb32, sem_b)
      cp_a.wait()
      cp_b.wait()
      pltpu.sync_copy(ra32, out_a.at[pl.ds(tail, 32)])
      pltpu.sync_copy(rb32, out_b.at[pl.ds(tail, 32)])

  return _sc_decoder_gather


def _tc_decode_body(a_ref, b_ref, b1_ref, w2t_ref, b2_ref, o_ref):
    z = jnp.maximum(a_ref[...] - b_ref[...] + b1_ref[...], 0.0)
    o_ref[...] = (jnp.dot(z, w2t_ref[...], preferred_element_type=jnp.float32)
                  + b2_ref[...])


_tc_decode = pl.pallas_call(
    _tc_decode_body,
    grid=(L_PAD // LW,),
    in_specs=[
        pl.BlockSpec((LW, D), lambda i: (i, 0)),
        pl.BlockSpec((LW, D), lambda i: (i, 0)),
        pl.BlockSpec((1, D), lambda i: (0, 0)),
        pl.BlockSpec((D, 1), lambda i: (0, 0)),
        pl.BlockSpec((1, 1), lambda i: (0, 0)),
    ],
    out_specs=pl.BlockSpec((LW, 1), lambda i: (i, 0)),
    out_shape=jax.ShapeDtypeStruct((L_PAD, 1), jnp.float32),
)


def kernel(x_virus, x_host, edge_index_vh, edge_index_hv, edge_label_index,
           Wl_vh, Wr_vh, b_vh, Wl_hv, Wr_hv, b_hv, W1, b1, W2, b2):
    svh, cvh, shv, chv = _build_sc_edge_agg()(
        x_virus, x_host,
        edge_index_vh[0], edge_index_vh[1],
        edge_index_hv[0], edge_index_hv[1],
    )

    zv1, zh1 = _tc_combine(
        svh, cvh, shv, chv, x_virus, x_host,
        Wl_vh.T, Wr_vh.T, b_vh.reshape(1, D),
        Wl_hv.T, Wr_hv.T, b_hv.reshape(1, D),
        W1.T,
    )

    ipad = jnp.zeros((L_PAD - L,), jnp.int32)
    row = jnp.concatenate([edge_label_index[0], ipad])
    col = jnp.concatenate([edge_label_index[1], ipad])
    a, b = _build_sc_decoder_gather()(zv1, zh1, row, col)

    out = _tc_decode(a, b, b1.reshape(1, D), W2.reshape(D, 1),
                     b2.reshape(1, 1))
    return out[:L, 0]


# speedup vs baseline: 1.3507x; 1.1343x over previous
import functools

import jax
import jax.numpy as jnp
from jax import lax
from jax.experimental import pallas as pl
from jax.experimental.pallas import tpu as pltpu
from jax.experimental.pallas import tpu_sc as plsc

N = 5000
D = 128
N_PAD = 5120
E = 160000
L = 50000
L_PAD = 50176
NW = 32
ET = E // 16
LW = L_PAD // NW


@functools.cache
def _get_mesh():
    return plsc.VectorSubcoreMesh(core_axis_name="c", subcore_axis_name="s")


@functools.cache
def _build_sc_edge_agg():
  kern = functools.partial(
    pl.kernel,
    out_type=(
        jax.ShapeDtypeStruct((N_PAD, D), jnp.float32),
        jax.ShapeDtypeStruct((16, N_PAD), jnp.int32),
        jax.ShapeDtypeStruct((N_PAD, D), jnp.float32),
        jax.ShapeDtypeStruct((16, N_PAD), jnp.int32),
    ),
    mesh=_get_mesh(),
    compiler_params=pltpu.CompilerParams(needs_layout_passes=False),
    scratch_types=[
        pltpu.VMEM_SHARED((N_PAD, D), jnp.float32),
        pltpu.VMEM((128,), jnp.int32),
        pltpu.VMEM((128,), jnp.int32),
        pltpu.VMEM((16,), jnp.int32),
        pltpu.VMEM((16,), jnp.int32),
        pltpu.VMEM((128, D), jnp.float32),
        pltpu.VMEM((16, D), jnp.float32),
        pltpu.VMEM((N_PAD,), jnp.int32),
        pltpu.VMEM((16, D), jnp.float32),
        pltpu.SemaphoreType.DMA,
    ],
  )

  @kern
  def _sc_edge_agg(xv_hbm, xh_hbm, svh, dvh, shv, dhv,
                   out_svh, out_cvh, out_shv, out_chv,
                   acc, idx_s, idx_d, idx_s16, idx_d16,
                   rows, rows16, hist, zbuf, sem):
      cid = lax.axis_index("c")
      sid = lax.axis_index("s")

      one16i = jnp.full((16,), 1, jnp.int32)
      zero16 = jnp.zeros((16,), jnp.float32)
      zero16i = jnp.zeros((16,), jnp.int32)

      @pl.loop(0, N_PAD // 16)
      def _zero_hist(i):
          hist[pl.ds(pl.multiple_of(i * 16, 16), 16)] = zero16i

      for r in range(16):
          for c8 in range(D // 16):
              zbuf[r, pl.ds(c8 * 16, 16)] = zero16
      row0 = sid * 320
      for k in range(20):
          pltpu.sync_copy(zbuf, acc.at[pl.ds(row0 + k * 16, 16)])
      plsc.subcore_barrier()

      def run_edges(x_hbm, s_hbm, d_hbm, cnt_out):
          base0 = sid * ET

          @pl.loop(0, 78)
          def _chunk(j):
              base = pl.multiple_of(base0 + j * 128, 8)
              pltpu.sync_copy(s_hbm.at[pl.ds(base, 128)], idx_s)
              pltpu.sync_copy(d_hbm.at[pl.ds(base, 128)], idx_d)
              pltpu.async_copy(x_hbm.at[idx_s], rows, sem).wait()
              pltpu.sync_copy(rows, acc.at[idx_d], add=True)
              for g in range(8):
                  plsc.addupdate_scatter(hist, [idx_d[pl.ds(g * 16, 16)]], one16i)

          tail = base0 + 78 * 128
          pltpu.sync_copy(s_hbm.at[pl.ds(tail, 16)], idx_s16)
          pltpu.sync_copy(d_hbm.at[pl.ds(tail, 16)], idx_d16)
          pltpu.async_copy(x_hbm.at[idx_s16], rows16, sem).wait()
          pltpu.sync_copy(rows16, acc.at[idx_d16], add=True)
          plsc.addupdate_scatter(hist, [idx_d16[...]], one16i)
          pltpu.sync_copy(hist, cnt_out.at[sid])

      @pl.when(cid == 0)
      def _():
          run_edges(xv_hbm, svh, dvh, out_cvh)

      @pl.when(cid == 1)
      def _():
          run_edges(xh_hbm, shv, dhv, out_chv)

      plsc.subcore_barrier()

      @pl.when(cid == 0)
      def _():
          pltpu.sync_copy(acc.at[pl.ds(row0, 320)], out_svh.at[pl.ds(row0, 320)])

      @pl.when(cid == 1)
      def _():
          pltpu.sync_copy(acc.at[pl.ds(row0, 320)], out_shv.at[pl.ds(row0, 320)])

  return _sc_edge_agg


def _tc_combine_body(svh_ref, cvh_ref, shv_ref, chv_ref, xv_ref, xh_ref,
                     wlt_vh_ref, wrt_vh_ref, bvh_ref,
                     wlt_hv_ref, wrt_hv_ref, bhv_ref,
                     w1t_ref, zv1_ref, zh1_ref):
    ones_j = jnp.ones((16, D), jnp.float32)

    def node_update(sums, cnts, x_dst, wlt, wrt, b):
        cnt2d = lax.dot_general(cnts.astype(jnp.float32), ones_j,
                                (((0,), (0,)), ((), ())),
                                preferred_element_type=jnp.float32)
        mean = sums[:N] / jnp.maximum(cnt2d[:N], 1.0)
        h = (jnp.dot(mean, wlt, preferred_element_type=jnp.float32)
             + b
             + jnp.dot(x_dst, wrt, preferred_element_type=jnp.float32))
        return jnp.maximum(h, 0.0)

    z_host = node_update(svh_ref[...], cvh_ref[...], xh_ref[...],
                         wlt_vh_ref[...], wrt_vh_ref[...], bvh_ref[...])
    z_virus = node_update(shv_ref[...], chv_ref[...], xv_ref[...],
                          wlt_hv_ref[...], wrt_hv_ref[...], bhv_ref[...])
    w1t = w1t_ref[...]
    zv1_ref[...] = jnp.dot(z_virus, w1t, preferred_element_type=jnp.float32)
    zh1_ref[...] = jnp.dot(z_host, w1t, preferred_element_type=jnp.float32)


_tc_combine = pl.pallas_call(
    _tc_combine_body,
    out_shape=(
        jax.ShapeDtypeStruct((N, D), jnp.float32),
        jax.ShapeDtypeStruct((N, D), jnp.float32),
    ),
)


@functools.cache
def _build_sc_decoder_gather():
  kern = functools.partial(
    pl.kernel,
    out_type=(
        jax.ShapeDtypeStruct((L_PAD, D), jnp.float32),
        jax.ShapeDtypeStruct((L_PAD, D), jnp.float32),
    ),
    mesh=_get_mesh(),
    scratch_types=[
        pltpu.VMEM((128,), jnp.int32),
        pltpu.VMEM((128,), jnp.int32),
        pltpu.VMEM((32,), jnp.int32),
        pltpu.VMEM((32,), jnp.int32),
        pltpu.VMEM((128, D), jnp.float32),
        pltpu.VMEM((128, D), jnp.float32),
        pltpu.VMEM((32, D), jnp.float32),
        pltpu.VMEM((32, D), jnp.float32),
        pltpu.SemaphoreType.DMA,
        pltpu.SemaphoreType.DMA,
    ],
  )

  @kern
  def _sc_decoder_gather(zv1_hbm, zh1_hbm, row_hbm, col_hbm,
                         out_a, out_b,
                         idx_r, idx_c, idx_r32, idx_c32,
                         ra, rb, ra32, rb32, sem_a, sem_b):
      cid = lax.axis_index("c")
      sid = lax.axis_index("s")
      wid = sid * 2 + cid
      base0 = wid * LW

      @pl.loop(0, 12)
      def _chunk(j):
          base = pl.multiple_of(base0 + j * 128, 8)
          pltpu.sync_copy(row_hbm.at[pl.ds(base, 128)], idx_r)
          pltpu.sync_copy(col_hbm.at[pl.ds(base, 128)], idx_c)
          cp_a = pltpu.async_copy(zv1_hbm.at[idx_r], ra, sem_a)
          cp_b = pltpu.async_copy(zh1_hbm.at[idx_c], rb, sem_b)
          cp_a.wait()
          cp_b.wait()
          pltpu.sync_copy(ra, out_a.at[pl.ds(base, 128)])
          pltpu.sync_copy(rb, out_b.at[pl.ds(base, 128)])

      tail = base0 + 12 * 128
      pltpu.sync_copy(row_hbm.at[pl.ds(tail, 32)], idx_r32)
      pltpu.sync_copy(col_hbm.at[pl.ds(tail, 32)], idx_c32)
      cp_a = pltpu.async_copy(zv1_hbm.at[idx_r32], ra32, sem_a)
      cp_b = pltpu.async_copy(zh1_hbm.at[idx_c32], rb32, sem_b)
      cp_a.wait()
      cp_b.wait()
      pltpu.sync_copy(ra32, out_a.at[pl.ds(tail, 32)])
      pltpu.sync_copy(rb32, out_b.at[pl.ds(tail, 32)])

  return _sc_decoder_gather


def _tc_decode_body(a_ref, b_ref, b1_ref, w2t_ref, b2_ref, o_ref):
    z = jnp.maximum(a_ref[...] - b_ref[...] + b1_ref[...], 0.0)
    o_ref[...] = (jnp.dot(z, w2t_ref[...], preferred_element_type=jnp.float32)
                  + b2_ref[...])


_tc_decode = pl.pallas_call(
    _tc_decode_body,
    grid=(L_PAD // LW,),
    in_specs=[
        pl.BlockSpec((LW, D), lambda i: (i, 0)),
        pl.BlockSpec((LW, D), lambda i: (i, 0)),
        pl.BlockSpec((1, D), lambda i: (0, 0)),
        pl.BlockSpec((D, 1), lambda i: (0, 0)),
        pl.BlockSpec((1, 1), lambda i: (0, 0)),
    ],
    out_specs=pl.BlockSpec((LW, 1), lambda i: (i, 0)),
    out_shape=jax.ShapeDtypeStruct((L_PAD, 1), jnp.float32),
)


def kernel(x_virus, x_host, edge_index_vh, edge_index_hv, edge_label_index,
           Wl_vh, Wr_vh, b_vh, Wl_hv, Wr_hv, b_hv, W1, b1, W2, b2):
    svh, cvh, shv, chv = _build_sc_edge_agg()(
        x_virus, x_host,
        edge_index_vh[0], edge_index_vh[1],
        edge_index_hv[0], edge_index_hv[1],
    )

    zv1, zh1 = _tc_combine(
        svh, cvh, shv, chv, x_virus, x_host,
        Wl_vh.T, Wr_vh.T, b_vh.reshape(1, D),
        Wl_hv.T, Wr_hv.T, b_hv.reshape(1, D),
        W1.T,
    )

    ipad = jnp.zeros((L_PAD - L,), jnp.int32)
    row = jnp.concatenate([edge_label_index[0], ipad])
    col = jnp.concatenate([edge_label_index[1], ipad])
    a, b = _build_sc_decoder_gather()(zv1, zh1, row, col)

    out = _tc_decode(a, b, b1.reshape(1, D), W2.reshape(D, 1),
                     b2.reshape(1, 1))
    return out[:L, 0]
